# Initial kernel scaffold; baseline (speedup 1.0000x reference)
import jax
import jax.numpy as jnp
from jax.experimental import pallas as pl


def kernel(x, edge_index):
    raise NotImplementedError("write your pallas kernel here")


# speedup vs baseline: 6.5671x; 6.5671x over previous
import jax
import jax.numpy as jnp
from jax import lax
from jax.experimental import pallas as pl
from jax.experimental.pallas import tpu as pltpu
from jax.experimental.pallas import tpu_sc as plsc

N_NODES = 10000
D_FEAT = 128
N_EDGES = 320000
NC = 2
NS = 16
NW = NC * NS
EPW = N_EDGES // NW
K = 128
NFULL = EPW // K
REM = EPW - NFULL * K
ROWS_PER_TILE = (N_NODES // NS) // 8 * 8
ROWS_TAIL = N_NODES - NS * ROWS_PER_TILE


def _sc_partial_body(x_hbm, src_hbm, dst_hbm, p_hbm,
                     src_v, dst_v, rows_v, srcr_v, dstr_v, rowsr_v,
                     acc, sem):
    cid = lax.axis_index("c")
    sid = lax.axis_index("s")
    wid = cid * NS + sid
    base = wid * EPW

    r0 = sid * ROWS_PER_TILE
    pltpu.sync_copy(x_hbm.at[pl.ds(r0, ROWS_PER_TILE)],
                    acc.at[pl.ds(r0, ROWS_PER_TILE)])

    @pl.when(sid == NS - 1)
    def _init_tail():
        t0 = NS * ROWS_PER_TILE
        pltpu.sync_copy(x_hbm.at[pl.ds(t0, ROWS_TAIL)],
                        acc.at[pl.ds(t0, ROWS_TAIL)])

    plsc.subcore_barrier()

    def chunk(j, carry):
        off = base + j * K
        pltpu.sync_copy(src_hbm.at[pl.ds(off, K)], src_v)
        pltpu.sync_copy(dst_hbm.at[pl.ds(off, K)], dst_v)
        pltpu.async_copy(x_hbm.at[src_v], rows_v, sem).wait()
        pltpu.sync_copy(rows_v, acc.at[dst_v], add=True)
        return carry

    lax.fori_loop(0, NFULL, chunk, 0)

    if REM:
        off = base + NFULL * K
        pltpu.sync_copy(src_hbm.at[pl.ds(off, REM)], srcr_v)
        pltpu.sync_copy(dst_hbm.at[pl.ds(off, REM)], dstr_v)
        pltpu.async_copy(x_hbm.at[srcr_v], rowsr_v, sem).wait()
        pltpu.sync_copy(rowsr_v, acc.at[dstr_v], add=True)

    plsc.subcore_barrier()
    pltpu.sync_copy(acc.at[pl.ds(r0, ROWS_PER_TILE)],
                    p_hbm.at[cid, pl.ds(r0, ROWS_PER_TILE)])

    @pl.when(sid == NS - 1)
    def _drain_tail():
        t0 = NS * ROWS_PER_TILE
        pltpu.sync_copy(acc.at[pl.ds(t0, ROWS_TAIL)],
                        p_hbm.at[cid, pl.ds(t0, ROWS_TAIL)])


def _combine_body(x_ref, p_ref, o_ref):
    o_ref[...] = p_ref[0] + p_ref[1] - x_ref[...]


def kernel(x, edge_index):
    src = edge_index[0].astype(jnp.int32)
    dst = edge_index[1].astype(jnp.int32)

    mesh = plsc.VectorSubcoreMesh(core_axis_name="c", subcore_axis_name="s",
                                  num_cores=NC, num_subcores=NS)
    p = pl.kernel(
        _sc_partial_body,
        out_type=jax.ShapeDtypeStruct((NC, N_NODES, D_FEAT), jnp.float32),
        mesh=mesh,
        scratch_types=[
            pltpu.VMEM((K,), jnp.int32),
            pltpu.VMEM((K,), jnp.int32),
            pltpu.VMEM((K, D_FEAT), jnp.float32),
            pltpu.VMEM((REM,), jnp.int32),
            pltpu.VMEM((REM,), jnp.int32),
            pltpu.VMEM((REM, D_FEAT), jnp.float32),
            pltpu.VMEM_SHARED((N_NODES, D_FEAT), jnp.float32),
            pltpu.SemaphoreType.DMA,
        ],
    )(x, src, dst)

    BLK = 400
    out = pl.pallas_call(
        _combine_body,
        out_shape=jax.ShapeDtypeStruct((N_NODES, D_FEAT), jnp.float32),
        grid=(N_NODES // BLK,),
        in_specs=[pl.BlockSpec((BLK, D_FEAT), lambda i: (i, 0)),
                  pl.BlockSpec((NC, BLK, D_FEAT), lambda i: (0, i, 0))],
        out_specs=pl.BlockSpec((BLK, D_FEAT), lambda i: (i, 0)),
    )(x, p)
    return out
